# Initial kernel scaffold; baseline (speedup 1.0000x reference)
import math

import jax
import jax.numpy as jnp
from jax.experimental import pallas as pl

B = 4096
IN_DIM = 384
HID = 768
LAT = 64
NC = 8
CPC = 64
SF = LAT // 2
TILE = 512

_HI = jax.lax.Precision.HIGHEST


def _gelu(x):
    return jax.nn.gelu(x, approximate=False)


def _fused(x_ref, w1_ref, b1_ref, w2_ref, b2_ref, wv_ref, bv_ref, cc_ref,
           cb_ref, ws1_ref, bs1_ref, ws2_ref, bs2_ref,
           kchart_ref, kcode_ref, zn_ref, ztex_ref, rw_ref, zgeo_ref,
           vq_ref, idx_ref, znac_ref, cbar_ref):
    x = x_ref[...]
    h = _gelu(jnp.dot(x, w1_ref[...], precision=_HI) + b1_ref[...])
    h = _gelu(jnp.dot(h, w2_ref[...], precision=_HI) + b2_ref[...])
    v = jnp.dot(h, wv_ref[...], precision=_HI) + bv_ref[...]

    cc = cc_ref[...]
    scores = jnp.dot(v, cc.T, precision=_HI) * (1.0 / math.sqrt(LAT))
    smax = jnp.max(scores, axis=1, keepdims=True)
    e = jnp.exp(scores - smax)
    rw = e / jnp.sum(e, axis=1, keepdims=True)
    k_chart = jnp.argmax(scores, axis=1, keepdims=True)

    c_bar = jnp.dot(rw, cc, precision=_HI)
    v_local = v - c_bar

    cb = cb_ref[...]
    dots = jnp.dot(v_local, cb.T, precision=_HI)
    code_sq = jnp.sum(cb * cb, axis=1, keepdims=True).T
    d2 = code_sq - 2.0 * dots

    iota_k = jax.lax.broadcasted_iota(jnp.int32, (x.shape[0], CPC), 1)
    zq_blend = jnp.zeros_like(v_local)
    zn_blend = jnp.zeros_like(v_local)
    k_code = jnp.zeros_like(k_chart)
    loss = jnp.zeros((), jnp.float32)
    ws1 = ws1_ref[...]
    bs1 = bs1_ref[...]
    ws2 = ws2_ref[...]
    bs2 = bs2_ref[...]
    for c in range(NC):
        cb_c = cb[c * CPC:(c + 1) * CPC, :]
        idx_c = jnp.argmin(d2[:, c * CPC:(c + 1) * CPC], axis=1,
                           keepdims=True)
        oh_c = (iota_k == idx_c).astype(jnp.float32)
        zq_c = jnp.dot(oh_c, cb_c, precision=_HI)
        delta_c = v_local - zq_c
        se_c = jnp.sum(delta_c * delta_c, axis=1)
        zn_c = jnp.dot(_gelu(jnp.dot(delta_c, ws1, precision=_HI) + bs1),
                       ws2, precision=_HI) + bs2
        rw_c = rw[:, c:c + 1]
        zq_blend = zq_blend + rw_c * zq_c
        zn_blend = zn_blend + rw_c * zn_c
        loss = loss + jnp.sum(rw[:, c] * se_c)
        k_code = k_code + jnp.where(k_chart == c, idx_c, 0)
        idx_ref[:, c:c + 1] = idx_c
        znac_ref[:, c, :] = zn_c

    kchart_ref[...] = k_chart
    kcode_ref[...] = k_code
    rw_ref[...] = rw
    cbar_ref[...] = c_bar
    zn_ref[...] = zn_blend
    ztex_ref[...] = (v_local - zq_blend) - zn_blend
    zgeo_ref[...] = c_bar + zq_blend + zn_blend

    contrib = loss * (1.25 / (B * LAT))

    @pl.when(pl.program_id(0) == 0)
    def _():
        vq_ref[0, 0] = 0.0

    vq_ref[0, 0] += contrib


def kernel(x, W1, b1, W2, b2, Wv, bv, chart_centers, codebook, Ws1, bs1,
           Ws2, bs2):
    nt = B // TILE
    row = lambda i: (i, 0)
    fixed = lambda i: (0, 0)
    out_shapes = (
        jax.ShapeDtypeStruct((B, 1), jnp.int32),
        jax.ShapeDtypeStruct((B, 1), jnp.int32),
        jax.ShapeDtypeStruct((B, LAT), jnp.float32),
        jax.ShapeDtypeStruct((B, LAT), jnp.float32),
        jax.ShapeDtypeStruct((B, NC), jnp.float32),
        jax.ShapeDtypeStruct((B, LAT), jnp.float32),
        jax.ShapeDtypeStruct((1, 1), jnp.float32),
        jax.ShapeDtypeStruct((B, NC), jnp.int32),
        jax.ShapeDtypeStruct((B, NC, LAT), jnp.float32),
        jax.ShapeDtypeStruct((B, LAT), jnp.float32),
    )
    out_specs = (
        pl.BlockSpec((TILE, 1), row),
        pl.BlockSpec((TILE, 1), row),
        pl.BlockSpec((TILE, LAT), row),
        pl.BlockSpec((TILE, LAT), row),
        pl.BlockSpec((TILE, NC), row),
        pl.BlockSpec((TILE, LAT), row),
        pl.BlockSpec((1, 1), fixed),
        pl.BlockSpec((TILE, NC), row),
        pl.BlockSpec((TILE, NC, LAT), lambda i: (i, 0, 0)),
        pl.BlockSpec((TILE, LAT), row),
    )
    in_specs = [
        pl.BlockSpec((TILE, IN_DIM), row),
        pl.BlockSpec((IN_DIM, HID), fixed),
        pl.BlockSpec((1, HID), fixed),
        pl.BlockSpec((HID, HID), fixed),
        pl.BlockSpec((1, HID), fixed),
        pl.BlockSpec((HID, LAT), fixed),
        pl.BlockSpec((1, LAT), fixed),
        pl.BlockSpec((NC, LAT), fixed),
        pl.BlockSpec((NC * CPC, LAT), fixed),
        pl.BlockSpec((LAT, SF), fixed),
        pl.BlockSpec((1, SF), fixed),
        pl.BlockSpec((SF, LAT), fixed),
        pl.BlockSpec((1, LAT), fixed),
    ]
    outs = pl.pallas_call(
        _fused,
        grid=(nt,),
        in_specs=in_specs,
        out_specs=out_specs,
        out_shape=out_shapes,
    )(x, W1, b1.reshape(1, HID), W2, b2.reshape(1, HID), Wv,
      bv.reshape(1, LAT), chart_centers, codebook.reshape(NC * CPC, LAT),
      Ws1, bs1.reshape(1, SF), Ws2, bs2.reshape(1, LAT))
    (k_chart, k_code, z_n, z_tex, rw, z_geo, vq, idx, znac, c_bar) = outs
    return (k_chart[:, 0], k_code[:, 0], z_n, z_tex, rw, z_geo,
            vq[0, 0], idx, znac, c_bar)


# speedup vs baseline: 1.9982x; 1.9982x over previous
import math

import jax
import jax.numpy as jnp
from jax.experimental import pallas as pl

B = 4096
IN_DIM = 384
HID = 768
LAT = 64
NC = 8
CPC = 64
SF = LAT // 2
TILE = 512


def _gelu(x):
    return 0.5 * x * (1.0 + jax.lax.erf(x * (1.0 / math.sqrt(2.0))))


def _fused(x_ref, w1_ref, b1_ref, w2_ref, b2_ref, wv_ref, bv_ref, cc_ref,
           cb_ref, ws1_ref, bs1_ref, ws2_ref, bs2_ref,
           kchart_ref, kcode_ref, zn_ref, ztex_ref, rw_ref, zgeo_ref,
           vq_ref, idx_ref, znac_ref, cbar_ref):
    x = x_ref[...]
    h = _gelu(jnp.dot(x, w1_ref[...]) + b1_ref[...])
    h = _gelu(jnp.dot(h, w2_ref[...]) + b2_ref[...])
    v = jnp.dot(h, wv_ref[...]) + bv_ref[...]

    cc = cc_ref[...]
    scores = jnp.dot(v, cc.T) * (1.0 / math.sqrt(LAT))
    smax = jnp.max(scores, axis=1, keepdims=True)
    e = jnp.exp(scores - smax)
    rw = e / jnp.sum(e, axis=1, keepdims=True)
    k_chart = jnp.argmax(scores, axis=1, keepdims=True)

    c_bar = jnp.dot(rw, cc)
    v_local = v - c_bar

    cb = cb_ref[...]
    iota_k = jax.lax.broadcasted_iota(jnp.int32, (x.shape[0], CPC), 1)
    zq_blend = jnp.zeros_like(v_local)
    zn_blend = jnp.zeros_like(v_local)
    k_code = jnp.zeros_like(k_chart)
    loss = jnp.zeros((1, 1), jnp.float32)
    ws1 = ws1_ref[...]
    bs1 = bs1_ref[...]
    ws2 = ws2_ref[...]
    bs2 = bs2_ref[...]
    for c in range(NC):
        cb_c = cb[c * CPC:(c + 1) * CPC, :]
        diff_c = v_local[:, None, :] - cb_c[None, :, :]
        dist_c = jnp.sum(diff_c * diff_c, axis=-1)
        idx_c = jnp.argmin(dist_c, axis=1, keepdims=True)
        oh_c = (iota_k == idx_c).astype(jnp.float32)
        zq_c = jnp.dot(oh_c, cb_c)
        delta_c = v_local - zq_c
        se_c = jnp.sum(delta_c * delta_c, axis=1, keepdims=True)
        zn_c = jnp.dot(_gelu(jnp.dot(delta_c, ws1) + bs1),
                       ws2) + bs2
        rw_c = rw[:, c:c + 1]
        zq_blend = zq_blend + rw_c * zq_c
        zn_blend = zn_blend + rw_c * zn_c
        loss = loss + jnp.sum(rw_c * se_c, axis=(0, 1), keepdims=True)
        k_code = k_code + jnp.where(k_chart == c, idx_c, 0)
        idx_ref[:, c:c + 1] = idx_c
        znac_ref[:, c, :] = zn_c

    kchart_ref[...] = k_chart
    kcode_ref[...] = k_code
    rw_ref[...] = rw
    cbar_ref[...] = c_bar
    zn_ref[...] = zn_blend
    ztex_ref[...] = (v_local - zq_blend) - zn_blend
    zgeo_ref[...] = c_bar + zq_blend + zn_blend

    contrib = loss * (1.25 / (B * LAT))

    @pl.when(pl.program_id(0) == 0)
    def _():
        vq_ref[...] = jnp.zeros((1, 1), jnp.float32)

    vq_ref[...] += contrib


def kernel(x, W1, b1, W2, b2, Wv, bv, chart_centers, codebook, Ws1, bs1,
           Ws2, bs2):
    nt = B // TILE
    row = lambda i: (i, 0)
    fixed = lambda i: (0, 0)
    out_shapes = (
        jax.ShapeDtypeStruct((B, 1), jnp.int32),
        jax.ShapeDtypeStruct((B, 1), jnp.int32),
        jax.ShapeDtypeStruct((B, LAT), jnp.float32),
        jax.ShapeDtypeStruct((B, LAT), jnp.float32),
        jax.ShapeDtypeStruct((B, NC), jnp.float32),
        jax.ShapeDtypeStruct((B, LAT), jnp.float32),
        jax.ShapeDtypeStruct((1, 1), jnp.float32),
        jax.ShapeDtypeStruct((B, NC), jnp.int32),
        jax.ShapeDtypeStruct((B, NC, LAT), jnp.float32),
        jax.ShapeDtypeStruct((B, LAT), jnp.float32),
    )
    out_specs = (
        pl.BlockSpec((TILE, 1), row),
        pl.BlockSpec((TILE, 1), row),
        pl.BlockSpec((TILE, LAT), row),
        pl.BlockSpec((TILE, LAT), row),
        pl.BlockSpec((TILE, NC), row),
        pl.BlockSpec((TILE, LAT), row),
        pl.BlockSpec((1, 1), fixed),
        pl.BlockSpec((TILE, NC), row),
        pl.BlockSpec((TILE, NC, LAT), lambda i: (i, 0, 0)),
        pl.BlockSpec((TILE, LAT), row),
    )
    in_specs = [
        pl.BlockSpec((TILE, IN_DIM), row),
        pl.BlockSpec((IN_DIM, HID), fixed),
        pl.BlockSpec((1, HID), fixed),
        pl.BlockSpec((HID, HID), fixed),
        pl.BlockSpec((1, HID), fixed),
        pl.BlockSpec((HID, LAT), fixed),
        pl.BlockSpec((1, LAT), fixed),
        pl.BlockSpec((NC, LAT), fixed),
        pl.BlockSpec((NC * CPC, LAT), fixed),
        pl.BlockSpec((LAT, SF), fixed),
        pl.BlockSpec((1, SF), fixed),
        pl.BlockSpec((SF, LAT), fixed),
        pl.BlockSpec((1, LAT), fixed),
    ]
    outs = pl.pallas_call(
        _fused,
        grid=(nt,),
        in_specs=in_specs,
        out_specs=out_specs,
        out_shape=out_shapes,
    )(x, W1, b1.reshape(1, HID), W2, b2.reshape(1, HID), Wv,
      bv.reshape(1, LAT), chart_centers, codebook.reshape(NC * CPC, LAT),
      Ws1, bs1.reshape(1, SF), Ws2, bs2.reshape(1, LAT))
    (k_chart, k_code, z_n, z_tex, rw, z_geo, vq, idx, znac, c_bar) = outs
    return (k_chart[:, 0], k_code[:, 0], z_n, z_tex, rw, z_geo,
            vq[0, 0], idx, znac, c_bar)
